# baseline (speedup 1.0000x reference)
import functools

import jax
import jax.numpy as jnp
from jax import lax
from jax.experimental import pallas as pl
from jax.experimental.pallas import tpu as pltpu
from jax.experimental.pallas import tpu_sc as plsc

NUM_FEAT = 1000000
NF = 26
H = 64
BATCH = 4096
NC, NS, L = 2, 16, 16
NW = NC * NS
BPW = BATCH // NW
NV = H // L
WP = 2 * H
MW = 128


def _take16(v, idx):
    return jnp.take_along_axis(v, idx, axis=0, mode="promise_in_bounds")


def _emb_body(xT_hbm, w2_hbm, m8_hbm, cb_hbm, out_hbm,
              idx_v, idxw_v, idxm_v, cb_v, wr0, wr1, mr0, mr1, ob0, ob1,
              gs0, gs1, osem):
    wid = lax.axis_index("s") * NC + lax.axis_index("c")
    b0 = wid * BPW
    pltpu.sync_copy(xT_hbm.at[:, pl.ds(b0, BPW)], idx_v)
    pltpu.sync_copy(cb_hbm, cb_v)

    lanes = lax.iota(jnp.int32, L)
    shamt = (lanes & 3) * 8
    widx = [lanes // 4 + 4 * k for k in range(NV)]

    wrs, mrs, obs, gss = (wr0, wr1), (mr0, mr1), (ob0, ob1), (gs0, gs1)

    def fill_idx(f):
        for j in range(BPW // L):
            iv = idx_v[f, pl.ds(L * j, L)]
            idxw_v[f, pl.ds(L * j, L)] = iv >> 1
            idxm_v[f, pl.ds(L * j, L)] = iv >> 3

    def start_gather(f):
        s = f % 2
        dw = pltpu.async_copy(w2_hbm.at[idxw_v.at[f]], wrs[s], gss[s])
        dm = pltpu.async_copy(m8_hbm.at[idxm_v.at[f]], mrs[s], gss[s])
        return dw, dm

    def compute(f):
        s = f % 2
        wrb, mrb, ob = wrs[s], mrs[s], obs[s]
        cbv = [cb_v[f, pl.ds(L * k, L)] for k in range(NV)]

        def row(b, c):
            iv16 = idx_v[f, pl.ds((b >> 4) << 4, L)]
            lane = jnp.full((L,), b & (L - 1), jnp.int32)
            rb = jnp.full((L,), b, jnp.int32)
            pf = _take16(iv16 & 1, lane).astype(jnp.float32)
            ov = _take16((iv16 & 7) << 4, lane)
            for k in range(NV):
                mw = plsc.load_gather(mrb, [rb, ov + widx[k]])
                mf = ((mw >> shamt) & 1).astype(jnp.float32)
                wlo = wrb[b, pl.ds(L * k, L)]
                whi = wrb[b, pl.ds(H + L * k, L)]
                wv = wlo + pf * (whi - wlo)
                ob[b, pl.ds(L * k, L)] = wv + mf * (cbv[k] - wv)
            return c

        lax.fori_loop(0, BPW, row, 0)

    fill_idx(0)
    pend = start_gather(0)
    for f in range(NF):
        if f + 1 < NF:
            fill_idx(f + 1)
            nxt = start_gather(f + 1)
        pend[0].wait()
        pend[1].wait()
        if f >= 2:
            pltpu.make_async_copy(
                obs[f % 2], out_hbm.at[pl.ds(b0, BPW), f - 2], osem).wait()
        compute(f)
        pltpu.async_copy(obs[f % 2], out_hbm.at[pl.ds(b0, BPW), f], osem)
        if f + 1 < NF:
            pend = nxt
    pltpu.make_async_copy(
        obs[(NF - 2) % 2], out_hbm.at[pl.ds(b0, BPW), NF - 2], osem).wait()
    pltpu.make_async_copy(
        obs[(NF - 1) % 2], out_hbm.at[pl.ds(b0, BPW), NF - 1], osem).wait()


@jax.jit
def _emb_call(xT, w2, m8, codebook):
    mesh = plsc.VectorSubcoreMesh(
        core_axis_name="c", subcore_axis_name="s")
    f = functools.partial(
        pl.kernel,
        out_type=jax.ShapeDtypeStruct((BATCH, NF, H), jnp.float32),
        mesh=mesh,
        scratch_types=[
            pltpu.VMEM((NF, BPW), jnp.int32),
            pltpu.VMEM((NF, BPW), jnp.int32),
            pltpu.VMEM((NF, BPW), jnp.int32),
            pltpu.VMEM((NF, H), jnp.float32),
            pltpu.VMEM((BPW, WP), jnp.float32),
            pltpu.VMEM((BPW, WP), jnp.float32),
            pltpu.VMEM((BPW, MW), jnp.int32),
            pltpu.VMEM((BPW, MW), jnp.int32),
            pltpu.VMEM((BPW, H), jnp.float32),
            pltpu.VMEM((BPW, H), jnp.float32),
            pltpu.SemaphoreType.DMA,
            pltpu.SemaphoreType.DMA,
            pltpu.SemaphoreType.DMA,
        ],
        compiler_params=pltpu.CompilerParams(needs_layout_passes=False),
    )(_emb_body)
    return f(xT, w2, m8, codebook)


def kernel(x, weight, codebook_mask, codebook):
    xT = x.T
    w2 = weight.reshape(NUM_FEAT // 2, WP)
    m8 = lax.bitcast_convert_type(
        codebook_mask.view(jnp.uint8).reshape(NUM_FEAT // 8, MW, 4),
        jnp.int32)
    return _emb_call(xT, w2, m8, codebook)


# speedup vs baseline: 3.3605x; 3.3605x over previous
import functools

import jax
import jax.numpy as jnp
from jax import lax
from jax.experimental import pallas as pl
from jax.experimental.pallas import tpu as pltpu
from jax.experimental.pallas import tpu_sc as plsc

NUM_FEAT = 1000000
NF = 26
H = 64
BATCH = 4096
NC, NS, L = 2, 16, 16
NW = NC * NS
BPW = BATCH // NW
NV = H // L
WP = 2 * H
MW = 128


def _take16(v, idx):
    return jnp.take_along_axis(v, idx, axis=0, mode="promise_in_bounds")


def _emb_body(xT_hbm, w2_hbm, m8_hbm, cb_hbm, out_hbm,
              idx_v, idxw_v, idxm_v, cb_v, wr0, wr1, mr0, mr1, ob0, ob1,
              gs0, gs1, osem):
    wid = lax.axis_index("s") * NC + lax.axis_index("c")
    b0 = wid * BPW
    pltpu.sync_copy(xT_hbm.at[:, pl.ds(b0, BPW)], idx_v)
    pltpu.sync_copy(cb_hbm, cb_v)

    lanes = lax.iota(jnp.int32, L)
    shamt = (lanes & 3) * 8
    widx = [lanes // 4 + 4 * k for k in range(NV)]

    wrs, mrs, obs, gss = (wr0, wr1), (mr0, mr1), (ob0, ob1), (gs0, gs1)

    def fill_idx(f):
        for j in range(BPW // L):
            iv = idx_v[f, pl.ds(L * j, L)]
            idxw_v[f, pl.ds(L * j, L)] = iv >> 1
            idxm_v[f, pl.ds(L * j, L)] = iv >> 3

    def start_gather(f):
        s = f % 2
        dw = pltpu.async_copy(w2_hbm.at[idxw_v.at[f]], wrs[s], gss[s])
        dm = pltpu.async_copy(m8_hbm.at[idxm_v.at[f]], mrs[s], gss[s])
        return dw, dm

    def compute(f):
        s = f % 2
        wrb, mrb, ob = wrs[s], mrs[s], obs[s]
        cbv = [cb_v[f, pl.ds(L * k, L)] for k in range(NV)]

        def row(b, c):
            iv16 = idx_v[f, pl.ds((b >> 4) << 4, L)]
            lane = jnp.full((L,), b & (L - 1), jnp.int32)
            rb = jnp.full((L,), b, jnp.int32)
            pf = _take16(iv16 & 1, lane).astype(jnp.float32)
            ov = _take16((iv16 & 7) << 4, lane)
            for k in range(NV):
                mw = plsc.load_gather(mrb, [rb, ov + widx[k]])
                mf = ((mw >> shamt) & 1).astype(jnp.float32)
                wlo = wrb[b, pl.ds(L * k, L)]
                whi = wrb[b, pl.ds(H + L * k, L)]
                wv = wlo + pf * (whi - wlo)
                ob[b, pl.ds(L * k, L)] = wv + mf * (cbv[k] - wv)
            return c

        lax.fori_loop(0, BPW, row, 0)

    fill_idx(0)
    pend = start_gather(0)
    for f in range(NF):
        if f + 1 < NF:
            fill_idx(f + 1)
            nxt = start_gather(f + 1)
        pend[0].wait()
        pend[1].wait()
        if f >= 2:
            pltpu.make_async_copy(
                obs[f % 2], out_hbm.at[pl.ds(b0, BPW), f - 2], osem).wait()
        compute(f)
        pltpu.async_copy(obs[f % 2], out_hbm.at[pl.ds(b0, BPW), f], osem)
        if f + 1 < NF:
            pend = nxt
    pltpu.make_async_copy(
        obs[(NF - 2) % 2], out_hbm.at[pl.ds(b0, BPW), NF - 2], osem).wait()
    pltpu.make_async_copy(
        obs[(NF - 1) % 2], out_hbm.at[pl.ds(b0, BPW), NF - 1], osem).wait()


@jax.jit
def _emb_call(xT, w2, m8, codebook):
    mesh = plsc.VectorSubcoreMesh(
        core_axis_name="c", subcore_axis_name="s")
    f = functools.partial(
        pl.kernel,
        out_type=jax.ShapeDtypeStruct((BATCH, NF, H), jnp.float32),
        mesh=mesh,
        scratch_types=[
            pltpu.VMEM((NF, BPW), jnp.int32),
            pltpu.VMEM((NF, BPW), jnp.int32),
            pltpu.VMEM((NF, BPW), jnp.int32),
            pltpu.VMEM((NF, H), jnp.float32),
            pltpu.VMEM((BPW, WP), jnp.float32),
            pltpu.VMEM((BPW, WP), jnp.float32),
            pltpu.VMEM((BPW, MW), jnp.int32),
            pltpu.VMEM((BPW, MW), jnp.int32),
            pltpu.VMEM((BPW, H), jnp.float32),
            pltpu.VMEM((BPW, H), jnp.float32),
            pltpu.SemaphoreType.DMA,
            pltpu.SemaphoreType.DMA,
            pltpu.SemaphoreType.DMA,
        ],
        compiler_params=pltpu.CompilerParams(needs_layout_passes=False),
    )(_emb_body)
    return f(xT, w2, m8, codebook)


def kernel(x, weight, codebook_mask, codebook):
    xT = x.T
    w2 = weight.reshape(NUM_FEAT // 2, WP)
    mT = codebook_mask.T
    mwT = (mT[0::4].astype(jnp.int32)
           | (mT[1::4].astype(jnp.int32) << 8)
           | (mT[2::4].astype(jnp.int32) << 16)
           | (mT[3::4].astype(jnp.int32) << 24))
    m8 = mwT.T.reshape(NUM_FEAT // 8, MW)
    return _emb_call(xT, w2, m8, codebook)


# speedup vs baseline: 3.4590x; 1.0293x over previous
import functools

import jax
import jax.numpy as jnp
from jax import lax
from jax.experimental import pallas as pl
from jax.experimental.pallas import tpu as pltpu
from jax.experimental.pallas import tpu_sc as plsc

NUM_FEAT = 1000000
NF = 26
H = 64
BATCH = 4096
NC, NS, L = 2, 16, 16
NW = NC * NS
BPW = BATCH // NW
NV = H // L
WP = 2 * H
MW = 128


def _take16(v, idx):
    return jnp.take_along_axis(v, idx, axis=0, mode="promise_in_bounds")


def _emb_body(xT_hbm, w2_hbm, mb_hbm, cb_hbm, out_hbm,
              idx_v, idxw_v, idxm_v, cb_v, wr0, wr1, mr0, mr1, ob0, ob1,
              gs0, gs1, osem):
    wid = lax.axis_index("s") * NC + lax.axis_index("c")
    b0 = wid * BPW
    pltpu.sync_copy(xT_hbm.at[:, pl.ds(b0, BPW)], idx_v)
    pltpu.sync_copy(cb_hbm, cb_v)

    lanes = lax.iota(jnp.int32, L)
    shsel = [lanes + 16 * (k % 2) for k in range(NV)]

    wrs, mrs, obs, gss = (wr0, wr1), (mr0, mr1), (ob0, ob1), (gs0, gs1)

    def fill_idx(f):
        for j in range(BPW // L):
            iv = idx_v[f, pl.ds(L * j, L)]
            idxw_v[f, pl.ds(L * j, L)] = iv >> 1
            idxm_v[f, pl.ds(L * j, L)] = iv >> 6

    def start_gather(f):
        s = f % 2
        dw = pltpu.async_copy(w2_hbm.at[idxw_v.at[f]], wrs[s], gss[s])
        dm = pltpu.async_copy(mb_hbm.at[idxm_v.at[f]], mrs[s], gss[s])
        return dw, dm

    def compute(f):
        s = f % 2
        wrb, mrb, ob = wrs[s], mrs[s], obs[s]
        cbv = [cb_v[f, pl.ds(L * k, L)] for k in range(NV)]

        def row(b, c):
            iv16 = idx_v[f, pl.ds((b >> 4) << 4, L)]
            lane = jnp.full((L,), b & (L - 1), jnp.int32)
            rb = jnp.full((L,), b, jnp.int32)
            pf = _take16(iv16 & 1, lane).astype(jnp.float32)
            ovb = _take16((iv16 & 63) << 1, lane)
            for k in range(NV):
                mwv = plsc.load_gather(mrb, [rb, ovb + (k // 2)])
                mf = ((mwv >> shsel[k]) & 1).astype(jnp.float32)
                wlo = wrb[b, pl.ds(L * k, L)]
                whi = wrb[b, pl.ds(H + L * k, L)]
                wv = wlo + pf * (whi - wlo)
                ob[b, pl.ds(L * k, L)] = wv + mf * (cbv[k] - wv)
            return c

        lax.fori_loop(0, BPW, row, 0)

    fill_idx(0)
    pend = start_gather(0)
    for f in range(NF):
        if f + 1 < NF:
            fill_idx(f + 1)
            nxt = start_gather(f + 1)
        pend[0].wait()
        pend[1].wait()
        if f >= 2:
            pltpu.make_async_copy(
                obs[f % 2], out_hbm.at[pl.ds(b0, BPW), f - 2], osem).wait()
        compute(f)
        pltpu.async_copy(obs[f % 2], out_hbm.at[pl.ds(b0, BPW), f], osem)
        if f + 1 < NF:
            pend = nxt
    pltpu.make_async_copy(
        obs[(NF - 2) % 2], out_hbm.at[pl.ds(b0, BPW), NF - 2], osem).wait()
    pltpu.make_async_copy(
        obs[(NF - 1) % 2], out_hbm.at[pl.ds(b0, BPW), NF - 1], osem).wait()


@jax.jit
def _emb_call(xT, w2, mb, codebook):
    mesh = plsc.VectorSubcoreMesh(
        core_axis_name="c", subcore_axis_name="s")
    f = functools.partial(
        pl.kernel,
        out_type=jax.ShapeDtypeStruct((BATCH, NF, H), jnp.float32),
        mesh=mesh,
        scratch_types=[
            pltpu.VMEM((NF, BPW), jnp.int32),
            pltpu.VMEM((NF, BPW), jnp.int32),
            pltpu.VMEM((NF, BPW), jnp.int32),
            pltpu.VMEM((NF, H), jnp.float32),
            pltpu.VMEM((BPW, WP), jnp.float32),
            pltpu.VMEM((BPW, WP), jnp.float32),
            pltpu.VMEM((BPW, MW), jnp.int32),
            pltpu.VMEM((BPW, MW), jnp.int32),
            pltpu.VMEM((BPW, H), jnp.float32),
            pltpu.VMEM((BPW, H), jnp.float32),
            pltpu.SemaphoreType.DMA,
            pltpu.SemaphoreType.DMA,
            pltpu.SemaphoreType.DMA,
        ],
        compiler_params=pltpu.CompilerParams(needs_layout_passes=False),
    )(_emb_body)
    return f(xT, w2, mb, codebook)


def kernel(x, weight, codebook_mask, codebook):
    xT = x.T
    w2 = weight.reshape(NUM_FEAT // 2, WP)
    mT = codebook_mask.T
    bits_lo = mT[0].astype(jnp.uint32)
    bits_hi = mT[32].astype(jnp.uint32)
    for j in range(1, 32):
        bits_lo = bits_lo | (mT[j].astype(jnp.uint32) << j)
        bits_hi = bits_hi | (mT[32 + j].astype(jnp.uint32) << j)
    mb = lax.bitcast_convert_type(
        jnp.stack([bits_lo, bits_hi], axis=-1),
        jnp.int32).reshape(NUM_FEAT // 64, MW)
    return _emb_call(xT, w2, mb, codebook)


# speedup vs baseline: 4.0822x; 1.1802x over previous
import functools

import jax
import jax.numpy as jnp
from jax import lax
from jax.experimental import pallas as pl
from jax.experimental.pallas import tpu as pltpu
from jax.experimental.pallas import tpu_sc as plsc

NUM_FEAT = 1000000
NF = 26
H = 64
BATCH = 4096
NC, NS, L = 2, 16, 16
NW = NC * NS
BPW = BATCH // NW
NV = H // L
WP = 2 * H
MW = 128


def _take16(v, idx):
    return jnp.take_along_axis(v, idx, axis=0, mode="promise_in_bounds")


def _emb_body(xT_hbm, w2_hbm, mb_hbm, cb_hbm, out_hbm,
              idx_v, idxw_v, idxm_v, cb_v, wr0, wr1, mr0, mr1, ob0, ob1,
              gs0, gs1, osem):
    wid = lax.axis_index("s") * NC + lax.axis_index("c")
    b0 = wid * BPW
    pltpu.sync_copy(xT_hbm.at[:, pl.ds(b0, BPW)], idx_v)
    pltpu.sync_copy(cb_hbm, cb_v)

    lanes = lax.iota(jnp.int32, L)
    shsel = [lanes + 16 * (k % 2) for k in range(NV)]

    wrs, mrs, obs, gss = (wr0, wr1), (mr0, mr1), (ob0, ob1), (gs0, gs1)

    def fill_idx(f):
        for j in range(BPW // L):
            iv = idx_v[f, pl.ds(L * j, L)]
            idxw_v[f, pl.ds(L * j, L)] = iv >> 1
            idxm_v[f, pl.ds(L * j, L)] = iv >> 6

    def start_gather(f):
        s = f % 2
        dw = pltpu.async_copy(w2_hbm.at[idxw_v.at[f]], wrs[s], gss[s])
        dm = pltpu.async_copy(mb_hbm.at[idxm_v.at[f]], mrs[s], gss[s])
        return dw, dm

    def compute(f):
        s = f % 2
        wrb, mrb, ob = wrs[s], mrs[s], obs[s]
        cbv = [cb_v[f, pl.ds(L * k, L)] for k in range(NV)]

        def row(b, c):
            iv16 = idx_v[f, pl.ds((b >> 4) << 4, L)]
            lane = jnp.full((L,), b & (L - 1), jnp.int32)
            rb = jnp.full((L,), b, jnp.int32)
            pf = _take16(iv16 & 1, lane).astype(jnp.float32)
            ovb = _take16((iv16 & 63) << 1, lane)
            for k in range(NV):
                mwv = plsc.load_gather(mrb, [rb, ovb + (k // 2)])
                mf = ((mwv >> shsel[k]) & 1).astype(jnp.float32)
                wlo = wrb[b, pl.ds(L * k, L)]
                whi = wrb[b, pl.ds(H + L * k, L)]
                wv = wlo + pf * (whi - wlo)
                ob[b, pl.ds(L * k, L)] = wv + mf * (cbv[k] - wv)
            return c

        lax.fori_loop(0, BPW, row, 0)

    fill_idx(0)
    pend = start_gather(0)
    for f in range(NF):
        if f + 1 < NF:
            fill_idx(f + 1)
            nxt = start_gather(f + 1)
        pend[0].wait()
        pend[1].wait()
        if f >= 2:
            pltpu.make_async_copy(
                obs[f % 2], out_hbm.at[pl.ds(b0, BPW), f - 2], osem).wait()
        compute(f)
        pltpu.async_copy(obs[f % 2], out_hbm.at[pl.ds(b0, BPW), f], osem)
        if f + 1 < NF:
            pend = nxt
    pltpu.make_async_copy(
        obs[(NF - 2) % 2], out_hbm.at[pl.ds(b0, BPW), NF - 2], osem).wait()
    pltpu.make_async_copy(
        obs[(NF - 1) % 2], out_hbm.at[pl.ds(b0, BPW), NF - 1], osem).wait()


@jax.jit
def _emb_call(xT, w2, mb, codebook):
    mesh = plsc.VectorSubcoreMesh(
        core_axis_name="c", subcore_axis_name="s")
    f = functools.partial(
        pl.kernel,
        out_type=jax.ShapeDtypeStruct((BATCH, NF, H), jnp.float32),
        mesh=mesh,
        scratch_types=[
            pltpu.VMEM((NF, BPW), jnp.int32),
            pltpu.VMEM((NF, BPW), jnp.int32),
            pltpu.VMEM((NF, BPW), jnp.int32),
            pltpu.VMEM((NF, H), jnp.float32),
            pltpu.VMEM((BPW, WP), jnp.float32),
            pltpu.VMEM((BPW, WP), jnp.float32),
            pltpu.VMEM((BPW, MW), jnp.int32),
            pltpu.VMEM((BPW, MW), jnp.int32),
            pltpu.VMEM((BPW, H), jnp.float32),
            pltpu.VMEM((BPW, H), jnp.float32),
            pltpu.SemaphoreType.DMA,
            pltpu.SemaphoreType.DMA,
            pltpu.SemaphoreType.DMA,
        ],
        compiler_params=pltpu.CompilerParams(needs_layout_passes=False),
    )(_emb_body)
    return f(xT, w2, mb, codebook)


def kernel(x, weight, codebook_mask, codebook):
    xT = x.T
    w2 = weight.reshape(NUM_FEAT // 2, WP)
    pmat = jnp.zeros((H, 4), jnp.float32).at[
        jnp.arange(H), jnp.arange(H) // 16].set(
        (2.0 ** (jnp.arange(H) % 16)).astype(jnp.float32))
    pack16 = lax.dot_general(
        codebook_mask.astype(jnp.float32), pmat,
        (((1,), (0,)), ((), ())),
        preferred_element_type=jnp.float32)
    u = pack16.astype(jnp.uint32)
    w0 = u[:, 0] | (u[:, 1] << 16)
    w1 = u[:, 2] | (u[:, 3] << 16)
    mb = lax.bitcast_convert_type(
        jnp.stack([w0, w1], axis=-1),
        jnp.int32).reshape(NUM_FEAT // 64, MW)
    return _emb_call(xT, w2, mb, codebook)


# speedup vs baseline: 9.7039x; 2.3771x over previous
import functools

import jax
import jax.numpy as jnp
from jax import lax
from jax.experimental import pallas as pl
from jax.experimental.pallas import tpu as pltpu
from jax.experimental.pallas import tpu_sc as plsc

NUM_FEAT = 1000000
NF = 26
H = 64
BATCH = 4096
NC, NS, L = 2, 16, 16
NW = NC * NS
BPW = BATCH // NW
CH = 64
NH = BPW // CH
NV = H // L
WP = 2 * H
MW = 128
NSLAB = (NUM_FEAT + MW - 1) // MW

_STEPS = [(f, h) for f in range(NF) for h in range(NH)]


def _take16(v, idx):
    return jnp.take_along_axis(v, idx, axis=0, mode="promise_in_bounds")


def _emb_body(xT_hbm, w2_hbm, mb0_hbm, mb1_hbm, cb_hbm, out_hbm,
              idx_v, idxw_v, idxm_v, cb_v, wr0, wr1,
              mr00, mr01, mr10, mr11, ob0, ob1,
              gs0, gs1, osem):
    wid = lax.axis_index("s") * NC + lax.axis_index("c")
    b0 = wid * BPW
    pltpu.sync_copy(xT_hbm.at[:, pl.ds(b0, BPW)], idx_v)
    pltpu.sync_copy(cb_hbm, cb_v)

    lanes = lax.iota(jnp.int32, L)
    shsel = [lanes + 16 * (k % 2) for k in range(NV)]

    wrs, obs, gss = (wr0, wr1), (ob0, ob1), (gs0, gs1)
    mrs = ((mr00, mr10), (mr01, mr11))

    def fill_idx(f):
        for j in range(BPW // L):
            iv = idx_v[f, pl.ds(L * j, L)]
            idxw_v[f, pl.ds(L * j, L)] = iv >> 1
            idxm_v[f, pl.ds(L * j, L)] = iv >> 7

    def start_gather(s):
        f, h = _STEPS[s]
        p = s % 2
        dw = pltpu.async_copy(
            w2_hbm.at[idxw_v.at[f, pl.ds(h * CH, CH)]], wrs[p], gss[p])
        d0 = pltpu.async_copy(
            mb0_hbm.at[idxm_v.at[f, pl.ds(h * CH, CH)]], mrs[p][0], gss[p])
        d1 = pltpu.async_copy(
            mb1_hbm.at[idxm_v.at[f, pl.ds(h * CH, CH)]], mrs[p][1], gss[p])
        return dw, d0, d1

    def out_slice(s):
        f, h = _STEPS[s]
        return out_hbm.at[pl.ds(b0 + h * CH, CH), f]

    def compute(s):
        f, h = _STEPS[s]
        p = s % 2
        wrb, ob, mrb = wrs[p], obs[p], mrs[p]
        base = h * CH
        cbv = [cb_v[f, pl.ds(L * k, L)] for k in range(NV)]

        def row(b, c):
            iv16 = idx_v[f, pl.ds(base + ((b >> 4) << 4), L)]
            lane = jnp.full((L,), b & (L - 1), jnp.int32)
            rb = jnp.full((L,), b, jnp.int32)
            pf = _take16(iv16 & 1, lane).astype(jnp.float32)
            ovb = _take16(iv16 & (MW - 1), lane)
            for k in range(NV):
                mwv = plsc.load_gather(mrb[k // 2], [rb, ovb])
                mf = ((mwv >> shsel[k]) & 1).astype(jnp.float32)
                wlo = wrb[b, pl.ds(L * k, L)]
                whi = wrb[b, pl.ds(H + L * k, L)]
                wv = wlo + pf * (whi - wlo)
                ob[b, pl.ds(L * k, L)] = wv + mf * (cbv[k] - wv)
            return c

        lax.fori_loop(0, CH, row, 0)

    nsteps = len(_STEPS)
    for f in range(NF):
        fill_idx(f)
    pend = start_gather(0)
    for s in range(nsteps):
        if s + 1 < nsteps:
            nxt = start_gather(s + 1)
        for d in pend:
            d.wait()
        if s >= 2:
            pltpu.make_async_copy(obs[s % 2], out_slice(s - 2), osem).wait()
        compute(s)
        pltpu.async_copy(obs[s % 2], out_slice(s), osem)
        if s + 1 < nsteps:
            pend = nxt
    pltpu.make_async_copy(obs[(nsteps - 2) % 2], out_slice(nsteps - 2),
                          osem).wait()
    pltpu.make_async_copy(obs[(nsteps - 1) % 2], out_slice(nsteps - 1),
                          osem).wait()


@jax.jit
def _emb_call(xT, w2, mb0, mb1, codebook):
    mesh = plsc.VectorSubcoreMesh(
        core_axis_name="c", subcore_axis_name="s")
    f = functools.partial(
        pl.kernel,
        out_type=jax.ShapeDtypeStruct((BATCH, NF, H), jnp.float32),
        mesh=mesh,
        scratch_types=[
            pltpu.VMEM((NF, BPW), jnp.int32),
            pltpu.VMEM((NF, BPW), jnp.int32),
            pltpu.VMEM((NF, BPW), jnp.int32),
            pltpu.VMEM((NF, H), jnp.float32),
            pltpu.VMEM((CH, WP), jnp.float32),
            pltpu.VMEM((CH, WP), jnp.float32),
            pltpu.VMEM((CH, MW), jnp.int32),
            pltpu.VMEM((CH, MW), jnp.int32),
            pltpu.VMEM((CH, MW), jnp.int32),
            pltpu.VMEM((CH, MW), jnp.int32),
            pltpu.VMEM((CH, H), jnp.float32),
            pltpu.VMEM((CH, H), jnp.float32),
            pltpu.SemaphoreType.DMA,
            pltpu.SemaphoreType.DMA,
            pltpu.SemaphoreType.DMA,
        ],
        compiler_params=pltpu.CompilerParams(needs_layout_passes=False),
    )(_emb_body)
    return f(xT, w2, mb0, mb1, codebook)


def kernel(x, weight, codebook_mask, codebook):
    xT = x.T
    w2 = weight.reshape(NUM_FEAT // 2, WP)
    pmat = jnp.zeros((H, 4), jnp.float32).at[
        jnp.arange(H), jnp.arange(H) // 16].set(
        (2.0 ** (jnp.arange(H) % 16)).astype(jnp.float32))
    pack16 = lax.dot_general(
        codebook_mask.astype(jnp.float32), pmat,
        (((1,), (0,)), ((), ())),
        preferred_element_type=jnp.float32)
    u = pack16.astype(jnp.uint32)
    pad = NSLAB * MW - NUM_FEAT
    w0 = jnp.pad(u[:, 0] | (u[:, 1] << 16), (0, pad))
    w1 = jnp.pad(u[:, 2] | (u[:, 3] << 16), (0, pad))
    mb0 = lax.bitcast_convert_type(w0, jnp.int32).reshape(NSLAB, MW)
    mb1 = lax.bitcast_convert_type(w1, jnp.int32).reshape(NSLAB, MW)
    return _emb_call(xT, w2, mb0, mb1, codebook)
